# Initial kernel scaffold; baseline (speedup 1.0000x reference)
import jax
import jax.numpy as jnp
from jax.experimental import pallas as pl


def kernel(x, cos, sin, Wq, Wk, Wv, Wg, Wo, Wgate, W1, W2, Wsg, Wsk, bsk, Wsv, bsv):
    raise NotImplementedError("write your pallas kernel here")


# speedup vs baseline: 2.3604x; 2.3604x over previous
import functools

import jax
import jax.numpy as jnp
import numpy as np
from jax import lax
from jax.experimental import pallas as pl
from jax.experimental.pallas import tpu as pltpu
from jax.experimental.pallas import tpu_sc as plsc

_EPS = float(np.finfo(np.float32).eps)
_T, _C = 2048, 768
_H, _N = 12, 64
_D = _H * _N
_E, _K, _FF = 64, 2, 512
_NA = _T * _K
_BLK = 128
_NROWS = 4736
_BT = 256
_NTB = _T // _BT
_NSC = 32
_NPW = _NA // _NSC

_PREC = lax.Precision.DEFAULT


def _mm(a, b):
    return lax.dot_general(a, b, (((1,), (0,)), ((), ())),
                           precision=_PREC, preferred_element_type=jnp.float32)


def _mm_t(a, w):
    return lax.dot_general(a, w, (((1,), (1,)), ((), ())),
                           precision=_PREC, preferred_element_type=jnp.float32)


def _rmsn(x):
    return x * lax.rsqrt(jnp.mean(x * x, axis=-1, keepdims=True) + _EPS)


def _const_blockdiag_mean():
    bm = np.zeros((_D, _D), np.float32)
    for h in range(_H):
        bm[h * _N:(h + 1) * _N, h * _N:(h + 1) * _N] = 1.0 / _N
    return bm


def _const_rot():
    rm = np.zeros((_D, _D), np.float32)
    half = _N // 2
    for h in range(_H):
        b = h * _N
        for i in range(half):
            rm[b + half + i, b + i] = -1.0
            rm[b + i, b + half + i] = 1.0
    return rm


def _const_triu(n):
    return np.triu(np.ones((n, n), np.float32), 1)


def _const_tril(n):
    return np.tril(np.ones((n, n), np.float32), -1)


def _proj_kernel(x_ref, wq_ref, wk_ref, wv_ref, wg_ref, bm_ref, rm_ref,
                 cos_ref, sin_ref, q_ref, k_ref, v_ref, g_ref):
    xb = x_ref[...]
    xn = _rmsn(xb)
    q = _mm_t(xn, wq_ref[...])
    k = _mm_t(xn, wk_ref[...])
    v_ref[...] = _mm_t(xn, wv_ref[...])
    g_ref[...] = _mm_t(xn, wg_ref[...])
    bm = bm_ref[...]
    q = q * lax.rsqrt(_mm(q * q, bm) + _EPS)
    k = k * lax.rsqrt(_mm(k * k, bm) + _EPS)
    rm = rm_ref[...]
    cosb = jnp.concatenate([cos_ref[...]] * _H, axis=1)
    sinb = jnp.concatenate([sin_ref[...]] * _H, axis=1)
    q_ref[...] = q * cosb + _mm(q, rm) * sinb
    k_ref[...] = k * cosb + _mm(k, rm) * sinb


def _attn_kernel(q_ref, k_ref, v_ref, o_ref):
    t = pl.program_id(1)
    scale = jnp.float32(1.0 / np.sqrt(_N))
    rows = lax.broadcasted_iota(jnp.int32, (_BT, _BT), 0)
    cols = lax.broadcasted_iota(jnp.int32, (_BT, _BT), 1)
    outs = []
    for sub in range(2):
        lo, hi = sub * _N, (sub + 1) * _N
        qh = q_ref[:, lo:hi]

        def body(j, carry, lo=lo, hi=hi, qh=qh):
            m, l, acc = carry
            kb = k_ref[pl.ds(j * _BT, _BT), lo:hi]
            s = lax.dot_general(qh, kb, (((1,), (1,)), ((), ())),
                                precision=_PREC,
                                preferred_element_type=jnp.float32) * scale
            s = jnp.where((t * _BT + rows) >= (j * _BT + cols), s,
                          jnp.float32(-1e30))
            mj = jnp.maximum(m, jnp.max(s, axis=-1, keepdims=True))
            p = jnp.exp(s - mj)
            alpha = jnp.exp(m - mj)
            l = l * alpha + jnp.sum(p, axis=-1, keepdims=True)
            acc = acc * alpha + _mm(p, v_ref[pl.ds(j * _BT, _BT), lo:hi])
            return mj, l, acc

        m0 = jnp.full((_BT, 1), -1e30, jnp.float32)
        l0 = jnp.zeros((_BT, 1), jnp.float32)
        a0 = jnp.zeros((_BT, _N), jnp.float32)
        m, l, acc = lax.fori_loop(0, t + 1, body, (m0, l0, a0))
        outs.append(acc / l)
    o_ref[...] = jnp.concatenate(outs, axis=1)


def _router_kernel(x_ref, o_ref, g_ref, wo_ref, wgate_ref,
                   x2_ref, xf_ref, w_ref, i_ref):
    ob = o_ref[...] * jax.nn.sigmoid(g_ref[...])
    x2 = x_ref[...] + _mm_t(ob, wo_ref[...])
    x2_ref[...] = x2
    xf = _rmsn(x2)
    xf_ref[...] = xf
    logits = _mm_t(xf, wgate_ref[...])
    sm = jax.nn.softmax(logits, axis=-1)
    lane = lax.broadcasted_iota(jnp.int32, (_BT, _E), 1)
    m1 = jnp.max(sm, axis=-1, keepdims=True)
    i1 = jnp.min(jnp.where(sm == m1, lane, _E), axis=-1, keepdims=True)
    sm2 = jnp.where(lane == i1, jnp.float32(-1.0), sm)
    m2 = jnp.max(sm2, axis=-1, keepdims=True)
    i2 = jnp.min(jnp.where(sm2 == m2, lane, _E), axis=-1, keepdims=True)
    den = m1 + m2 + jnp.float32(1e-6)
    lane128 = lax.broadcasted_iota(jnp.int32, (_BT, 128), 1)
    w_ref[...] = jnp.where(lane128 == 0, m1 / den,
                           jnp.where(lane128 == 1, m2 / den, 0.0))
    i_ref[...] = jnp.where(lane128 == 0, i1, jnp.where(lane128 == 1, i2, 0))


def _sort_kernel(ea_ref, tri_ref, ltri_ref, pos_ref, meta_ref):
    tri = tri_ref[...]
    carry = jnp.zeros((_E, 1), jnp.float32)
    ranks = []
    for b in range(8):
        row = ea_ref[b:b + 1, :]
        oh = (jnp.broadcast_to(row, (_E, 512))
              == lax.broadcasted_iota(jnp.int32, (_E, 512), 0)
              ).astype(jnp.float32)
        cum = _mm(oh, tri) + carry
        ranks.append(jnp.sum(oh * cum, axis=0, keepdims=True))
        carry = carry + jnp.sum(oh, axis=1, keepdims=True)
    cnt = carry
    cnt8 = jnp.floor((cnt + 7.0) * 0.125) * 8.0
    off = _mm(ltri_ref[...], cnt8)
    nch = jnp.floor((cnt + jnp.float32(_BLK - 1)) * (1.0 / _BLK))
    lane = lax.broadcasted_iota(jnp.int32, (_E, 128), 1)
    meta_ref[...] = jnp.where(lane == 0, off, jnp.where(lane == 1, nch, 0.0)
                              ).astype(jnp.int32)
    for b in range(8):
        row = ea_ref[b:b + 1, :]
        oh = (jnp.broadcast_to(row, (_E, 512))
              == lax.broadcasted_iota(jnp.int32, (_E, 512), 0)
              ).astype(jnp.float32)
        offg = jnp.sum(oh * off, axis=0, keepdims=True)
        pos_ref[b:b + 1, :] = (ranks[b] + offg).astype(jnp.int32)


def _ffn_kernel(off_ref, nch_ref, xs_ref, w1_ref, w2_ref, ys_ref):
    e = pl.program_id(0)
    off = off_ref[e]
    nch = nch_ref[e]
    w1 = w1_ref[0]
    w2 = w2_ref[0]

    def body(i, _):
        st = pl.multiple_of(off + i * _BLK, 8)
        xb = xs_ref[pl.ds(st, _BLK), :]
        h = jnp.square(jnp.maximum(_mm_t(xb, w1), 0.0))
        ys_ref[pl.ds(st, _BLK), :] = _mm_t(h, w2)
        return 0

    lax.fori_loop(0, nch, body, 0)


def _shared_kernel(xf_ref, wsk_ref, wsv_ref, bsk_ref, bsv_ref, wsg_ref,
                   sh_ref):
    xf = xf_ref[...]
    xs = _rmsn(xf)
    hs = _mm_t(xs, wsk_ref[...]) + bsk_ref[0:1, :]
    hs = jnp.square(jnp.maximum(hs, 0.0))
    shared = _mm_t(hs, wsv_ref[...]) + bsv_ref[0:1, :]
    sg = jax.nn.sigmoid(_mm_t(xf, wsg_ref[...]))
    sh_ref[...] = sg[:, 0:1] * shared


def _final_kernel(x2_ref, y0_ref, y1_ref, w_ref, sh_ref, out_ref):
    w = w_ref[...]
    out_ref[...] = (x2_ref[...]
                    + w[:, 0:1] * y0_ref[...]
                    + w[:, 1:2] * y1_ref[...]
                    + sh_ref[...])


def _sc_dispatch(xf, posj):
    mesh = plsc.VectorSubcoreMesh(core_axis_name="c", subcore_axis_name="s")

    @functools.partial(
        pl.kernel,
        out_type=jax.ShapeDtypeStruct((_NROWS, _C), jnp.float32),
        mesh=mesh,
        scratch_types=[pltpu.VMEM((_NPW,), jnp.int32),
                       pltpu.VMEM((_NPW, _C), jnp.float32),
                       pltpu.SemaphoreType.DMA],
    )
    def k(xf_hbm, pos_hbm, xs_hbm, pos_v, rows_v, sem):
        wid = lax.axis_index("s") * 2 + lax.axis_index("c")
        base = wid * _NPW
        tokbase = lax.rem(base, _T)
        pltpu.sync_copy(pos_hbm.at[pl.ds(base, _NPW)], pos_v)
        pltpu.sync_copy(xf_hbm.at[pl.ds(tokbase, _NPW)], rows_v)
        pltpu.async_copy(rows_v, xs_hbm.at[pos_v], sem).wait()

    return k(xf, posj)


def _sc_collect(ys, posj):
    mesh = plsc.VectorSubcoreMesh(core_axis_name="c", subcore_axis_name="s")

    @functools.partial(
        pl.kernel,
        out_type=[jax.ShapeDtypeStruct((_T, _C), jnp.float32),
                  jax.ShapeDtypeStruct((_T, _C), jnp.float32)],
        mesh=mesh,
        scratch_types=[pltpu.VMEM((_NPW,), jnp.int32),
                       pltpu.VMEM((_NPW, _C), jnp.float32),
                       pltpu.SemaphoreType.DMA],
    )
    def k(ys_hbm, pos_hbm, y0_hbm, y1_hbm, pos_v, rows_v, sem):
        wid = lax.axis_index("s") * 2 + lax.axis_index("c")
        base = wid * _NPW
        tokbase = lax.rem(base, _T)
        pltpu.sync_copy(pos_hbm.at[pl.ds(base, _NPW)], pos_v)
        pltpu.async_copy(ys_hbm.at[pos_v], rows_v, sem).wait()

        @pl.when(wid < _NSC // 2)
        def _():
            pltpu.sync_copy(rows_v, y0_hbm.at[pl.ds(tokbase, _NPW)])

        @pl.when(wid >= _NSC // 2)
        def _():
            pltpu.sync_copy(rows_v, y1_hbm.at[pl.ds(tokbase, _NPW)])

    return k(ys, posj)


def kernel(x, cos, sin, Wq, Wk, Wv, Wg, Wo, Wgate, W1, W2, Wsg, Wsk, bsk,
           Wsv, bsv):
    xm = x.reshape(_T, _C)
    f32 = jnp.float32

    bm = jnp.asarray(_const_blockdiag_mean())
    rm = jnp.asarray(_const_rot())
    tri = jnp.asarray(_const_triu(512))
    ltri = jnp.asarray(_const_tril(_E))

    q, k, v, g = pl.pallas_call(
        _proj_kernel,
        grid=(_NTB,),
        in_specs=[
            pl.BlockSpec((_BT, _C), lambda t: (t, 0)),
            pl.BlockSpec((_D, _C), lambda t: (0, 0)),
            pl.BlockSpec((_D, _C), lambda t: (0, 0)),
            pl.BlockSpec((_D, _C), lambda t: (0, 0)),
            pl.BlockSpec((_D, _C), lambda t: (0, 0)),
            pl.BlockSpec((_D, _D), lambda t: (0, 0)),
            pl.BlockSpec((_D, _D), lambda t: (0, 0)),
            pl.BlockSpec((_BT, _N), lambda t: (t, 0)),
            pl.BlockSpec((_BT, _N), lambda t: (t, 0)),
        ],
        out_specs=[pl.BlockSpec((_BT, _D), lambda t: (t, 0))] * 4,
        out_shape=[jax.ShapeDtypeStruct((_T, _D), f32)] * 4,
    )(xm, Wq, Wk, Wv, Wg, bm, rm, cos, sin)

    o = pl.pallas_call(
        _attn_kernel,
        grid=(_H // 2, _NTB),
        in_specs=[
            pl.BlockSpec((_BT, 2 * _N), lambda h, t: (t, h)),
            pl.BlockSpec((_T, 2 * _N), lambda h, t: (0, h)),
            pl.BlockSpec((_T, 2 * _N), lambda h, t: (0, h)),
        ],
        out_specs=pl.BlockSpec((_BT, 2 * _N), lambda h, t: (t, h)),
        out_shape=jax.ShapeDtypeStruct((_T, _D), f32),
    )(q, k, v)

    x2, xf, wts, idxs = pl.pallas_call(
        _router_kernel,
        grid=(_NTB,),
        in_specs=[
            pl.BlockSpec((_BT, _C), lambda t: (t, 0)),
            pl.BlockSpec((_BT, _D), lambda t: (t, 0)),
            pl.BlockSpec((_BT, _D), lambda t: (t, 0)),
            pl.BlockSpec((_C, _D), lambda t: (0, 0)),
            pl.BlockSpec((_E, _C), lambda t: (0, 0)),
        ],
        out_specs=[
            pl.BlockSpec((_BT, _C), lambda t: (t, 0)),
            pl.BlockSpec((_BT, _C), lambda t: (t, 0)),
            pl.BlockSpec((_BT, 128), lambda t: (t, 0)),
            pl.BlockSpec((_BT, 128), lambda t: (t, 0)),
        ],
        out_shape=[
            jax.ShapeDtypeStruct((_T, _C), f32),
            jax.ShapeDtypeStruct((_T, _C), f32),
            jax.ShapeDtypeStruct((_T, 128), f32),
            jax.ShapeDtypeStruct((_T, 128), jnp.int32),
        ],
    )(xm, o, g, Wo, Wgate)

    bsk2 = jnp.broadcast_to(bsk[None, :], (8, _FF))
    bsv2 = jnp.broadcast_to(bsv[None, :], (8, _C))
    wsgp = jnp.pad(Wsg, ((0, 127), (0, 0)))
    sh = pl.pallas_call(
        _shared_kernel,
        grid=(_NTB,),
        in_specs=[
            pl.BlockSpec((_BT, _C), lambda t: (t, 0)),
            pl.BlockSpec((_FF, _C), lambda t: (0, 0)),
            pl.BlockSpec((_C, _FF), lambda t: (0, 0)),
            pl.BlockSpec((8, _FF), lambda t: (0, 0)),
            pl.BlockSpec((8, _C), lambda t: (0, 0)),
            pl.BlockSpec((128, _C), lambda t: (0, 0)),
        ],
        out_specs=pl.BlockSpec((_BT, _C), lambda t: (t, 0)),
        out_shape=jax.ShapeDtypeStruct((_T, _C), f32),
    )(xf, Wsk, Wsv, bsk2, bsv2, wsgp)

    ea = jnp.concatenate([idxs[:, 0], idxs[:, 1]]).reshape(8, 512)
    pos8, meta = pl.pallas_call(
        _sort_kernel,
        grid=(1,),
        in_specs=[
            pl.BlockSpec((8, 512), lambda i: (0, 0)),
            pl.BlockSpec((512, 512), lambda i: (0, 0)),
            pl.BlockSpec((_E, _E), lambda i: (0, 0)),
        ],
        out_specs=[
            pl.BlockSpec((8, 512), lambda i: (0, 0)),
            pl.BlockSpec((_E, 128), lambda i: (0, 0)),
        ],
        out_shape=[
            jax.ShapeDtypeStruct((8, 512), jnp.int32),
            jax.ShapeDtypeStruct((_E, 128), jnp.int32),
        ],
    )(ea, tri, ltri)
    posj = pos8.reshape(_NA)
    off_arr = meta[:, 0]
    nch_arr = meta[:, 1]

    xsrt = _sc_dispatch(xf, posj)

    ysrt = pl.pallas_call(
        _ffn_kernel,
        grid_spec=pltpu.PrefetchScalarGridSpec(
            num_scalar_prefetch=2,
            grid=(_E,),
            in_specs=[
                pl.BlockSpec((_NROWS, _C), lambda e, so, sn: (0, 0)),
                pl.BlockSpec((1, _FF, _C), lambda e, so, sn: (e, 0, 0)),
                pl.BlockSpec((1, _C, _FF), lambda e, so, sn: (e, 0, 0)),
            ],
            out_specs=pl.BlockSpec((_NROWS, _C), lambda e, so, sn: (0, 0)),
        ),
        out_shape=jax.ShapeDtypeStruct((_NROWS, _C), f32),
    )(off_arr, nch_arr, xsrt, W1, W2)

    y0, y1 = _sc_collect(ysrt, posj)

    out = pl.pallas_call(
        _final_kernel,
        grid=(_NTB,),
        in_specs=[
            pl.BlockSpec((_BT, _C), lambda t: (t, 0)),
            pl.BlockSpec((_BT, _C), lambda t: (t, 0)),
            pl.BlockSpec((_BT, _C), lambda t: (t, 0)),
            pl.BlockSpec((_BT, 128), lambda t: (t, 0)),
            pl.BlockSpec((_BT, _C), lambda t: (t, 0)),
        ],
        out_specs=pl.BlockSpec((_BT, _C), lambda t: (t, 0)),
        out_shape=jax.ShapeDtypeStruct((_T, _C), f32),
    )(x2, y0, y1, wts, sh)

    return out.reshape(1, _T, _C)
